# baseline (speedup 1.0000x reference)
import functools

import jax
import jax.numpy as jnp
from jax import lax
from jax.experimental import pallas as pl
from jax.experimental.pallas import tpu as pltpu

B, D, A, S = 16384, 128, 64, 64
BLK = 1024


def _fused_body(state_ref, wt_ref, bias_ref, mean_ref, std_ref, value_ref, onehot_ref):
    x = state_ref[...]
    res = jnp.dot(x, wt_ref[...]) + bias_ref[...]
    mean_ref[...] = res[:, :A]
    std_ref[...] = jnp.zeros_like(res[:, :A])
    value_ref[...] = res[:, A + S:A + S + 1]
    logits = res[:, A:A + S]
    idx = jnp.argmax(logits, axis=1)
    onehot_ref[...] = (
        lax.broadcasted_iota(jnp.int32, (BLK, S), 1) == idx[:, None]
    ).astype(jnp.float32)


@functools.partial(jax.jit, static_argnames=())
def kernel(state, W_skill, b_skill, W_action, b_action, W_value, b_value):
    wt = jnp.concatenate(
        [W_action.T, W_skill.T, W_value.T,
         jnp.zeros((D, 127), jnp.float32)], axis=1)
    bias = jnp.concatenate(
        [b_action, b_skill, b_value, jnp.zeros((127,), jnp.float32)])[None, :]

    grid = (B // BLK,)
    mean, std, value, one_hot = pl.pallas_call(
        _fused_body,
        grid=grid,
        in_specs=[
            pl.BlockSpec((BLK, D), lambda i: (i, 0)),
            pl.BlockSpec((D, 256), lambda i: (0, 0)),
            pl.BlockSpec((1, 256), lambda i: (0, 0)),
        ],
        out_specs=[
            pl.BlockSpec((BLK, A), lambda i: (i, 0)),
            pl.BlockSpec((BLK, A), lambda i: (i, 0)),
            pl.BlockSpec((BLK, 1), lambda i: (i, 0)),
            pl.BlockSpec((BLK, S), lambda i: (i, 0)),
        ],
        out_shape=[
            jax.ShapeDtypeStruct((B, A), jnp.float32),
            jax.ShapeDtypeStruct((B, A), jnp.float32),
            jax.ShapeDtypeStruct((B, 1), jnp.float32),
            jax.ShapeDtypeStruct((B, S), jnp.float32),
        ],
        compiler_params=pltpu.CompilerParams(
            dimension_semantics=("arbitrary",),
        ),
    )(state, wt, bias)
    return (mean, std, value[:, 0], one_hot)


# speedup vs baseline: 1.1402x; 1.1402x over previous
import functools

import jax
import jax.numpy as jnp
from jax import lax
from jax.experimental import pallas as pl
from jax.experimental.pallas import tpu as pltpu

B, D, A, S = 16384, 128, 64, 64
BLK = 4096


def _fused_body(state_ref, wt_ref, bias_ref, mean_ref, std_ref, value_ref, onehot_ref):
    x = state_ref[...]
    res = jnp.dot(x, wt_ref[...]) + bias_ref[...]
    mean_ref[...] = res[:, :A]
    std_ref[...] = jnp.zeros_like(res[:, :A])
    value_ref[...] = res[:, A + S:A + S + 1]
    logits = res[:, A:A + S]
    idx = jnp.argmax(logits, axis=1)
    onehot_ref[...] = (
        lax.broadcasted_iota(jnp.int32, (BLK, S), 1) == idx[:, None]
    ).astype(jnp.float32)


@functools.partial(jax.jit, static_argnames=())
def kernel(state, W_skill, b_skill, W_action, b_action, W_value, b_value):
    wt = jnp.concatenate(
        [W_action.T, W_skill.T, W_value.T,
         jnp.zeros((D, 127), jnp.float32)], axis=1)
    bias = jnp.concatenate(
        [b_action, b_skill, b_value, jnp.zeros((127,), jnp.float32)])[None, :]

    grid = (B // BLK,)
    mean, std, value, one_hot = pl.pallas_call(
        _fused_body,
        grid=grid,
        in_specs=[
            pl.BlockSpec((BLK, D), lambda i: (i, 0)),
            pl.BlockSpec((D, 256), lambda i: (0, 0)),
            pl.BlockSpec((1, 256), lambda i: (0, 0)),
        ],
        out_specs=[
            pl.BlockSpec((BLK, A), lambda i: (i, 0)),
            pl.BlockSpec((BLK, A), lambda i: (i, 0)),
            pl.BlockSpec((BLK, 1), lambda i: (i, 0)),
            pl.BlockSpec((BLK, S), lambda i: (i, 0)),
        ],
        out_shape=[
            jax.ShapeDtypeStruct((B, A), jnp.float32),
            jax.ShapeDtypeStruct((B, A), jnp.float32),
            jax.ShapeDtypeStruct((B, 1), jnp.float32),
            jax.ShapeDtypeStruct((B, S), jnp.float32),
        ],
        compiler_params=pltpu.CompilerParams(
            dimension_semantics=("arbitrary",),
        ),
    )(state, wt, bias)
    return (mean, std, value[:, 0], one_hot)
